# baseline (speedup 1.0000x reference)
import functools

import jax
import jax.numpy as jnp
from jax import lax
from jax.experimental import pallas as pl
from jax.experimental.pallas import tpu as pltpu
from jax.experimental.pallas import tpu_sc as plsc

T = 3
N = 10000
E = 320000
D = 128
HW = D // 2
NP = 10240
NC = 2
NS = 16
RB = 1000
NB = N // RB
CH = 128
EPP = 327680
EPT = EPP // NS
NCH = EPT // CH
IB = 1
NBLK = NCH // IB
NPS = NP // NS
WBR = 40


@functools.cache
def _get_mesh():
    return plsc.VectorSubcoreMesh(core_axis_name="c", subcore_axis_name="s",
                                  num_cores=NC, num_subcores=NS)


def _prop_body(y2, src2d, dst2d, o2, srcb, dstb, rows, wb, acc, semg, sems):
    c = lax.axis_index("c")
    s = lax.axis_index("s")
    row0 = c * NP + s * NPS

    def load_and_fire(b, p):
        r0 = s * NCH + b * IB
        pltpu.sync_copy(src2d.at[pl.ds(c * (NS * NCH) + r0, IB)],
                        srcb.at[p])
        pltpu.sync_copy(dst2d.at[pl.ds(c * (NS * NCH) + r0, IB)],
                        dstb.at[p])
        for k in range(IB):
            pltpu.async_copy(y2.at[srcb.at[p, k]],
                             rows.at[p, pl.ds(k * CH, CH)], semg)

    def drain_gathers(p):
        for k in range(IB):
            pltpu.make_async_copy(y2.at[srcb.at[p, k]],
                                  rows.at[p, pl.ds(k * CH, CH)],
                                  semg).wait()

    def fire_scatters(p):
        for k in range(IB):
            pltpu.async_copy(rows.at[p, pl.ds(k * CH, CH)],
                             acc.at[dstb.at[p, k]], sems, add=True)

    def drain_scatters(p):
        for k in range(IB):
            pltpu.make_async_copy(rows.at[p, pl.ds(k * CH, CH)],
                                  acc.at[dstb.at[p, k]], sems).wait()

    def initp(i, carry):
        pltpu.sync_copy(y2.at[pl.ds(row0 + i * WBR, WBR)], wb)
        pltpu.sync_copy(wb, acc.at[pl.ds(s * NPS + i * WBR, WBR)])
        return carry

    lax.fori_loop(0, NPS // WBR, initp, 0)
    load_and_fire(0, 0)
    plsc.subcore_barrier()

    def block(b, carry):
        p = lax.rem(b, 2)
        q = 1 - p
        drain_gathers(p)

        @pl.when(b > 0)
        def _():
            drain_scatters(q)

        @pl.when(b < NBLK - 1)
        def _():
            load_and_fire(b + 1, q)

        fire_scatters(p)
        return carry

    lax.fori_loop(0, NBLK, block, 0)
    drain_scatters((NBLK - 1) % 2)
    plsc.subcore_barrier()

    def outp(i, carry):
        pltpu.sync_copy(acc.at[pl.ds(s * NPS + i * WBR, WBR)], wb)
        pltpu.sync_copy(wb, o2.at[pl.ds(row0 + i * WBR, WBR)])
        return carry

    lax.fori_loop(0, NPS // WBR, outp, 0)


@functools.cache
def _get_prop():
    return pl.kernel(
        _prop_body,
        out_type=jax.ShapeDtypeStruct((2 * NP, D), jnp.float32),
        mesh=_get_mesh(),
        scratch_types=[
            pltpu.VMEM((2, IB, CH), jnp.int32),
            pltpu.VMEM((2, IB, CH), jnp.int32),
            pltpu.VMEM((2, IB * CH, D), jnp.float32),
            pltpu.VMEM((WBR, D), jnp.float32),
            pltpu.VMEM_SHARED((NP, D), jnp.float32),
            pltpu.SemaphoreType.DMA,
            pltpu.SemaphoreType.DMA,
        ],
        compiler_params=pltpu.CompilerParams(use_tc_tiling_on_sc=False),
    )


def _prop(y2, src2, dst):
    return _get_prop()(y2, src2, dst)


def _mm_body(x_ref, w_ref, o_ref):
    o_ref[...] = jnp.dot(x_ref[...], w_ref[...],
                         preferred_element_type=jnp.float32)


def _xproj(xs2, wxc):
    return pl.pallas_call(
        _mm_body,
        grid=(T * N // RB,),
        in_specs=[
            pl.BlockSpec((RB, D), lambda i: (i, 0)),
            pl.BlockSpec((D, 3 * D), lambda i: (0, 0)),
        ],
        out_specs=pl.BlockSpec((RB, 3 * D), lambda i: (i, 0)),
        out_shape=jax.ShapeDtypeStruct((T * N, 3 * D), jnp.float32),
    )(xs2, wxc)


def _dinv_body(dg_ref, o_ref):
    o_ref[...] = lax.rsqrt(dg_ref[0][:, 0:1])


def _dinv_t(o2deg, slab):
    return pl.pallas_call(
        _dinv_body,
        grid=(NB,),
        in_specs=[pl.BlockSpec((1, RB, D), lambda i, slab=slab: (slab, i, 0))],
        out_specs=pl.BlockSpec((RB, 1), lambda i: (i, 0)),
        out_shape=jax.ShapeDtypeStruct((N, 1), jnp.float32),
    )(o2deg)


def _zr_body(xp_ref, h_ref, wh_ref, di_ref, o_ref):
    y = (xp_ref[...] + jnp.dot(h_ref[...], wh_ref[...],
                               preferred_element_type=jnp.float32)
         ) * di_ref[...]
    o_ref[...] = jnp.stack([y[:, :D], y[:, D:]], axis=0)


def _zr_pre(xp, h, whzr, dinv_t, t):
    return pl.pallas_call(
        _zr_body,
        grid=(NB,),
        in_specs=[
            pl.BlockSpec((RB, 2 * D), lambda i, t=t: (t * NB + i, 0)),
            pl.BlockSpec((RB, D), lambda i: (i, 0)),
            pl.BlockSpec((D, 2 * D), lambda i: (0, 0)),
            pl.BlockSpec((RB, 1), lambda i: (i, 0)),
        ],
        out_specs=pl.BlockSpec((2, RB, D), lambda i: (0, i, 0)),
        out_shape=jax.ShapeDtypeStruct((2, NP, D), jnp.float32),
    )(xp, h, whzr, dinv_t)


def _gates_body(pr_ref, di_ref, br_ref, xph_ref, h_ref, whh_ref, o_ref):
    di = di_ref[...]
    r = jax.nn.sigmoid(di * pr_ref[0] + br_ref[...])
    rh = r * h_ref[...]
    yh = (xph_ref[...] + jnp.dot(rh, whh_ref[...],
                                 preferred_element_type=jnp.float32)) * di
    o_ref[...] = jnp.stack([yh, yh], axis=0)


def _gates(o2zr, dinv_t, br, xp, h, whh, t):
    return pl.pallas_call(
        _gates_body,
        grid=(NB,),
        in_specs=[
            pl.BlockSpec((1, RB, D), lambda i: (1, i, 0)),
            pl.BlockSpec((RB, 1), lambda i: (i, 0)),
            pl.BlockSpec((1, D), lambda i: (0, 0)),
            pl.BlockSpec((RB, D), lambda i, t=t: (t * NB + i, 2)),
            pl.BlockSpec((RB, D), lambda i: (i, 0)),
            pl.BlockSpec((D, D), lambda i: (0, 0)),
        ],
        out_specs=pl.BlockSpec((2, RB, D), lambda i: (0, i, 0)),
        out_shape=jax.ShapeDtypeStruct((2, NP, D), jnp.float32),
    )(o2zr, dinv_t, br, xp, h, whh)


def _final_body(pz_ref, oh_ref, di_ref, bz_ref, bh_ref, h0_ref, o_ref):
    di = di_ref[...]
    z = jax.nn.sigmoid(di * pz_ref[0] + bz_ref[...])
    hh = jnp.tanh(di * oh_ref[0] + bh_ref[...])
    o_ref[...] = z * h0_ref[...] + (1.0 - z) * hh


def _final(o2zr, o2h, dinv_t, bz, bh, h0):
    return pl.pallas_call(
        _final_body,
        grid=(NB,),
        in_specs=[
            pl.BlockSpec((1, RB, D), lambda i: (0, i, 0)),
            pl.BlockSpec((1, RB, D), lambda i: (0, i, 0)),
            pl.BlockSpec((RB, 1), lambda i: (i, 0)),
            pl.BlockSpec((1, D), lambda i: (0, 0)),
            pl.BlockSpec((1, D), lambda i: (0, 0)),
            pl.BlockSpec((1, D), lambda i: (0, 0)),
        ],
        out_specs=pl.BlockSpec((RB, D), lambda i: (i, 0)),
        out_shape=jax.ShapeDtypeStruct((N, D), jnp.float32),
    )(o2zr, o2h, dinv_t, bz, bh, h0)


def kernel(xs, eis, W_xz, b_xz, W_hz, b_hz, W_xr, b_xr, W_hr, b_hr,
           W_xh, b_xh, W_hh, b_hh):
    xs2 = xs.reshape(T * N, D)
    pad = jnp.full((T, EPP - E), N, jnp.int32)
    src = jnp.concatenate([eis[:, 0, :], pad], axis=1)
    dst = jnp.concatenate([eis[:, 1, :], pad], axis=1)

    def pack(t0, t1):
        s2 = jnp.concatenate([src[t0], src[t1] + NP]).reshape(
            2 * NS * NCH, CH)
        d2 = jnp.concatenate([dst[t0], dst[t1]]).reshape(2 * NS * NCH, CH)
        return s2, d2

    wxc = jnp.concatenate([W_xz, W_xr, W_xh], axis=1)
    whzr = jnp.concatenate([W_hz, W_hr], axis=1)
    bz = (b_xz + b_hz).reshape(1, D)
    br = (b_xr + b_hr).reshape(1, D)
    bh = (b_xh + b_hh).reshape(1, D)

    xp = _xproj(xs2, wxc)
    ones_tab = jnp.ones((2 * NP, D), jnp.float32)
    sA, dA = pack(0, 1)
    sB, dB = pack(2, 2)
    degA = _prop(ones_tab, sA, dA).reshape(2, NP, D)
    degB = _prop(ones_tab, sB, dB).reshape(2, NP, D)
    dinvs = [_dinv_t(degA, 0), _dinv_t(degA, 1), _dinv_t(degB, 0)]
    packs = [pack(t, t) for t in range(T)]

    h = jnp.zeros((N, D), jnp.float32)
    h0 = jnp.zeros((1, D), jnp.float32)
    outs = []
    for t in range(T):
        st, dt = packs[t]
        dinv_t = dinvs[t]
        y2 = _zr_pre(xp, h, whzr, dinv_t, t)
        o2zr = _prop(y2.reshape(2 * NP, D), st, dt).reshape(2, NP, D)
        yh2 = _gates(o2zr, dinv_t, br, xp, h, W_hh, t)
        o2h = _prop(yh2.reshape(2 * NP, D), st, dt).reshape(2, NP, D)
        h = _final(o2zr, o2h, dinv_t, bz, bh, h0)
        h0 = h[0:1]
        outs.append(h)
    return jnp.stack(outs)


# speedup vs baseline: 1.4783x; 1.4783x over previous
import functools

import jax
import jax.numpy as jnp
from jax import lax
from jax.experimental import pallas as pl
from jax.experimental.pallas import tpu as pltpu
from jax.experimental.pallas import tpu_sc as plsc

T = 3
N = 10000
E = 320000
D = 128
HW = D // 2
NP = 10240
NC = 2
NS = 16
RB = 1000
NB = N // RB
CH = 128
EPP = 327680
EPT = EPP // NS
NCH = EPT // CH
IB = 5
NBLK = NCH // IB
NPS = NP // NS
WBR = 40


@functools.cache
def _get_mesh():
    return plsc.VectorSubcoreMesh(core_axis_name="c", subcore_axis_name="s",
                                  num_cores=NC, num_subcores=NS)


def _prop_body(y2, src2d, dst2d, o2, srcb, dstb, rows, wb, acc, semg, sems):
    c = lax.axis_index("c")
    s = lax.axis_index("s")
    row0 = c * NP + s * NPS

    def load_and_fire(b, p):
        r0 = s * NCH + b * IB
        pltpu.sync_copy(src2d.at[pl.ds(c * (NS * NCH) + r0, IB)],
                        srcb.at[p])
        pltpu.sync_copy(dst2d.at[pl.ds(c * (NS * NCH) + r0, IB)],
                        dstb.at[p])
        for k in range(IB):
            pltpu.async_copy(y2.at[srcb.at[p, k]],
                             rows.at[p, pl.ds(k * CH, CH)], semg)

    def drain_gathers(p):
        for k in range(IB):
            pltpu.make_async_copy(y2.at[srcb.at[p, k]],
                                  rows.at[p, pl.ds(k * CH, CH)],
                                  semg).wait()

    def fire_scatters(p):
        for k in range(IB):
            pltpu.async_copy(rows.at[p, pl.ds(k * CH, CH)],
                             acc.at[dstb.at[p, k]], sems, add=True)

    def drain_scatters(p):
        for k in range(IB):
            pltpu.make_async_copy(rows.at[p, pl.ds(k * CH, CH)],
                                  acc.at[dstb.at[p, k]], sems).wait()

    def initp(i, carry):
        pltpu.sync_copy(y2.at[pl.ds(row0 + i * WBR, WBR)], wb)
        pltpu.sync_copy(wb, acc.at[pl.ds(s * NPS + i * WBR, WBR)])
        return carry

    lax.fori_loop(0, NPS // WBR, initp, 0)
    load_and_fire(0, 0)
    plsc.subcore_barrier()

    def block(b, carry):
        p = lax.rem(b, 2)
        q = 1 - p
        drain_gathers(p)

        @pl.when(b > 0)
        def _():
            drain_scatters(q)

        @pl.when(b < NBLK - 1)
        def _():
            load_and_fire(b + 1, q)

        fire_scatters(p)
        return carry

    lax.fori_loop(0, NBLK, block, 0)
    drain_scatters((NBLK - 1) % 2)
    plsc.subcore_barrier()

    def outp(i, carry):
        pltpu.sync_copy(acc.at[pl.ds(s * NPS + i * WBR, WBR)], wb)
        pltpu.sync_copy(wb, o2.at[pl.ds(row0 + i * WBR, WBR)])
        return carry

    lax.fori_loop(0, NPS // WBR, outp, 0)


@functools.cache
def _get_prop():
    return pl.kernel(
        _prop_body,
        out_type=jax.ShapeDtypeStruct((2 * NP, HW), jnp.float32),
        mesh=_get_mesh(),
        scratch_types=[
            pltpu.VMEM((2, IB, CH), jnp.int32),
            pltpu.VMEM((2, IB, CH), jnp.int32),
            pltpu.VMEM((2, IB * CH, HW), jnp.float32),
            pltpu.VMEM((WBR, HW), jnp.float32),
            pltpu.VMEM_SHARED((NP, HW), jnp.float32),
            pltpu.SemaphoreType.DMA,
            pltpu.SemaphoreType.DMA,
        ],
        compiler_params=pltpu.CompilerParams(use_tc_tiling_on_sc=False),
    )


def _prop(y2, src2, dst):
    return _get_prop()(y2, src2, dst)


def _mm_body(x_ref, w_ref, o_ref):
    o_ref[...] = jnp.dot(x_ref[...], w_ref[...],
                         preferred_element_type=jnp.float32)


def _xproj(xs2, wxc):
    return pl.pallas_call(
        _mm_body,
        grid=(T * N // RB,),
        in_specs=[
            pl.BlockSpec((RB, D), lambda i: (i, 0)),
            pl.BlockSpec((D, 3 * D), lambda i: (0, 0)),
        ],
        out_specs=pl.BlockSpec((RB, 3 * D), lambda i: (i, 0)),
        out_shape=jax.ShapeDtypeStruct((T * N, 3 * D), jnp.float32),
    )(xs2, wxc)


def _dinv_body(dg_ref, o_ref):
    o_ref[...] = lax.rsqrt(dg_ref[0][:, 0:1])


def _dinv_t(o2deg, slab):
    return pl.pallas_call(
        _dinv_body,
        grid=(NB,),
        in_specs=[pl.BlockSpec((1, RB, HW), lambda i, slab=slab: (slab, i, 0))],
        out_specs=pl.BlockSpec((RB, 1), lambda i: (i, 0)),
        out_shape=jax.ShapeDtypeStruct((N, 1), jnp.float32),
    )(o2deg)


def _zr_body(xp_ref, h_ref, wh_ref, di_ref, oz_ref, or_ref):
    y = (xp_ref[...] + jnp.dot(h_ref[...], wh_ref[...],
                               preferred_element_type=jnp.float32)
         ) * di_ref[...]
    oz_ref[...] = jnp.stack([y[:, 0:HW], y[:, HW:D]], axis=0)
    or_ref[...] = jnp.stack([y[:, D:D + HW], y[:, D + HW:2 * D]], axis=0)


def _zr_pre(xp, h, whzr, dinv_t, t):
    return pl.pallas_call(
        _zr_body,
        grid=(NB,),
        in_specs=[
            pl.BlockSpec((RB, 2 * D), lambda i, t=t: (t * NB + i, 0)),
            pl.BlockSpec((RB, D), lambda i: (i, 0)),
            pl.BlockSpec((D, 2 * D), lambda i: (0, 0)),
            pl.BlockSpec((RB, 1), lambda i: (i, 0)),
        ],
        out_specs=[
            pl.BlockSpec((2, RB, HW), lambda i: (0, i, 0)),
            pl.BlockSpec((2, RB, HW), lambda i: (0, i, 0)),
        ],
        out_shape=[
            jax.ShapeDtypeStruct((2, NP, HW), jnp.float32),
            jax.ShapeDtypeStruct((2, NP, HW), jnp.float32),
        ],
    )(xp, h, whzr, dinv_t)


def _gates_body(pr0_ref, pr1_ref, di_ref, br_ref, xph_ref, h_ref, whh_ref,
                o_ref):
    di = di_ref[...]
    pr = jnp.concatenate([pr0_ref[0], pr1_ref[0]], axis=1)
    r = jax.nn.sigmoid(di * pr + br_ref[...])
    rh = r * h_ref[...]
    yh = (xph_ref[...] + jnp.dot(rh, whh_ref[...],
                                 preferred_element_type=jnp.float32)) * di
    o_ref[...] = jnp.stack([yh[:, :HW], yh[:, HW:]], axis=0)


def _gates(o2r, dinv_t, br, xp, h, whh, t):
    return pl.pallas_call(
        _gates_body,
        grid=(NB,),
        in_specs=[
            pl.BlockSpec((1, RB, HW), lambda i: (0, i, 0)),
            pl.BlockSpec((1, RB, HW), lambda i: (1, i, 0)),
            pl.BlockSpec((RB, 1), lambda i: (i, 0)),
            pl.BlockSpec((1, D), lambda i: (0, 0)),
            pl.BlockSpec((RB, D), lambda i, t=t: (t * NB + i, 2)),
            pl.BlockSpec((RB, D), lambda i: (i, 0)),
            pl.BlockSpec((D, D), lambda i: (0, 0)),
        ],
        out_specs=pl.BlockSpec((2, RB, HW), lambda i: (0, i, 0)),
        out_shape=jax.ShapeDtypeStruct((2, NP, HW), jnp.float32),
    )(o2r, o2r, dinv_t, br, xp, h, whh)


def _final_body(pz0_ref, pz1_ref, oh0_ref, oh1_ref, di_ref, bz_ref, bh_ref,
                h0_ref, o_ref):
    di = di_ref[...]
    pz = jnp.concatenate([pz0_ref[0], pz1_ref[0]], axis=1)
    z = jax.nn.sigmoid(di * pz + bz_ref[...])
    oh = jnp.concatenate([oh0_ref[0], oh1_ref[0]], axis=1)
    hh = jnp.tanh(di * oh + bh_ref[...])
    o_ref[...] = z * h0_ref[...] + (1.0 - z) * hh


def _final(o2z, o2h, dinv_t, bz, bh, h0):
    return pl.pallas_call(
        _final_body,
        grid=(NB,),
        in_specs=[
            pl.BlockSpec((1, RB, HW), lambda i: (0, i, 0)),
            pl.BlockSpec((1, RB, HW), lambda i: (1, i, 0)),
            pl.BlockSpec((1, RB, HW), lambda i: (0, i, 0)),
            pl.BlockSpec((1, RB, HW), lambda i: (1, i, 0)),
            pl.BlockSpec((RB, 1), lambda i: (i, 0)),
            pl.BlockSpec((1, D), lambda i: (0, 0)),
            pl.BlockSpec((1, D), lambda i: (0, 0)),
            pl.BlockSpec((1, D), lambda i: (0, 0)),
        ],
        out_specs=pl.BlockSpec((RB, D), lambda i: (i, 0)),
        out_shape=jax.ShapeDtypeStruct((N, D), jnp.float32),
    )(o2z, o2z, o2h, o2h, dinv_t, bz, bh, h0)


def kernel(xs, eis, W_xz, b_xz, W_hz, b_hz, W_xr, b_xr, W_hr, b_hr,
           W_xh, b_xh, W_hh, b_hh):
    xs2 = xs.reshape(T * N, D)
    pad = jnp.full((T, EPP - E), N, jnp.int32)
    src = jnp.concatenate([eis[:, 0, :], pad], axis=1)
    dst = jnp.concatenate([eis[:, 1, :], pad], axis=1)

    def pack(t0, t1):
        s2 = jnp.concatenate([src[t0], src[t1] + NP]).reshape(
            2 * NS * NCH, CH)
        d2 = jnp.concatenate([dst[t0], dst[t1]]).reshape(2 * NS * NCH, CH)
        return s2, d2

    wxc = jnp.concatenate([W_xz, W_xr, W_xh], axis=1)
    whzr = jnp.concatenate([W_hz, W_hr], axis=1)
    bz = (b_xz + b_hz).reshape(1, D)
    br = (b_xr + b_hr).reshape(1, D)
    bh = (b_xh + b_hh).reshape(1, D)

    xp = _xproj(xs2, wxc)
    ones_tab = jnp.ones((2 * NP, HW), jnp.float32)
    sA, dA = pack(0, 1)
    sB, dB = pack(2, 2)
    degA = _prop(ones_tab, sA, dA).reshape(2, NP, HW)
    degB = _prop(ones_tab, sB, dB).reshape(2, NP, HW)
    dinvs = [_dinv_t(degA, 0), _dinv_t(degA, 1), _dinv_t(degB, 0)]
    packs = [pack(t, t) for t in range(T)]

    h = jnp.zeros((N, D), jnp.float32)
    h0 = jnp.zeros((1, D), jnp.float32)
    outs = []
    for t in range(T):
        st, dt = packs[t]
        dinv_t = dinvs[t]
        yz, yr = _zr_pre(xp, h, whzr, dinv_t, t)
        o2z = _prop(yz.reshape(2 * NP, HW), st, dt).reshape(2, NP, HW)
        o2r = _prop(yr.reshape(2 * NP, HW), st, dt).reshape(2, NP, HW)
        yh2 = _gates(o2r, dinv_t, br, xp, h, W_hh, t)
        o2h = _prop(yh2.reshape(2 * NP, HW), st, dt).reshape(2, NP, HW)
        h = _final(o2z, o2h, dinv_t, bz, bh, h0)
        h0 = h[0:1]
        outs.append(h)
    return jnp.stack(outs)


# speedup vs baseline: 1.5104x; 1.0217x over previous
import functools

import jax
import jax.numpy as jnp
from jax import lax
from jax.experimental import pallas as pl
from jax.experimental.pallas import tpu as pltpu
from jax.experimental.pallas import tpu_sc as plsc

T = 3
N = 10000
E = 320000
D = 128
HW = D // 2
NP = 10240
NC = 2
NS = 16
RB = 1000
NB = N // RB
CH = 128
EPP = 327680
EPT = EPP // NS
NCH = EPT // CH
IB = 5
NBLK = NCH // IB
NPS = NP // NS
WBR = 40


@functools.cache
def _get_mesh():
    return plsc.VectorSubcoreMesh(core_axis_name="c", subcore_axis_name="s",
                                  num_cores=NC, num_subcores=NS)


def _prop_body(y2, src2d, dst2d, o2, srcb, dstb, rows, wb, acc, semg, sems):
    c = lax.axis_index("c")
    s = lax.axis_index("s")
    row0 = c * NP + s * NPS

    def load_and_fire(b, p):
        r0 = s * NCH + b * IB
        pltpu.sync_copy(src2d.at[pl.ds(c * (NS * NCH) + r0, IB)],
                        srcb.at[p])
        pltpu.sync_copy(dst2d.at[pl.ds(c * (NS * NCH) + r0, IB)],
                        dstb.at[p])
        for k in range(IB):
            pltpu.async_copy(y2.at[srcb.at[p, k]],
                             rows.at[p, pl.ds(k * CH, CH)], semg.at[p])

    def drain_gathers(p):
        for k in range(IB):
            pltpu.make_async_copy(y2.at[srcb.at[p, k]],
                                  rows.at[p, pl.ds(k * CH, CH)],
                                  semg.at[p]).wait()

    def fire_scatters(p):
        for k in range(IB):
            pltpu.async_copy(rows.at[p, pl.ds(k * CH, CH)],
                             acc.at[dstb.at[p, k]], sems.at[p], add=True)

    def drain_scatters(p):
        for k in range(IB):
            pltpu.make_async_copy(rows.at[p, pl.ds(k * CH, CH)],
                                  acc.at[dstb.at[p, k]], sems.at[p]).wait()

    def initp(i, carry):
        pltpu.sync_copy(y2.at[pl.ds(row0 + i * WBR, WBR)], wb)
        pltpu.sync_copy(wb, acc.at[pl.ds(s * NPS + i * WBR, WBR)])
        return carry

    lax.fori_loop(0, NPS // WBR, initp, 0)
    load_and_fire(0, 0)
    plsc.subcore_barrier()

    def block(b, carry):
        p = lax.rem(b, 2)
        q = 1 - p
        drain_gathers(p)
        fire_scatters(p)

        @pl.when(b > 0)
        def _():
            drain_scatters(q)

        @pl.when(b < NBLK - 1)
        def _():
            load_and_fire(b + 1, q)

        return carry

    lax.fori_loop(0, NBLK, block, 0)
    drain_scatters((NBLK - 1) % 2)
    plsc.subcore_barrier()

    def outp(i, carry):
        pltpu.sync_copy(acc.at[pl.ds(s * NPS + i * WBR, WBR)], wb)
        pltpu.sync_copy(wb, o2.at[pl.ds(row0 + i * WBR, WBR)])
        return carry

    lax.fori_loop(0, NPS // WBR, outp, 0)


@functools.cache
def _get_prop():
    return pl.kernel(
        _prop_body,
        out_type=jax.ShapeDtypeStruct((2 * NP, HW), jnp.float32),
        mesh=_get_mesh(),
        scratch_types=[
            pltpu.VMEM((2, IB, CH), jnp.int32),
            pltpu.VMEM((2, IB, CH), jnp.int32),
            pltpu.VMEM((2, IB * CH, HW), jnp.float32),
            pltpu.VMEM((WBR, HW), jnp.float32),
            pltpu.VMEM_SHARED((NP, HW), jnp.float32),
            pltpu.SemaphoreType.DMA((2,)),
            pltpu.SemaphoreType.DMA((2,)),
        ],
        compiler_params=pltpu.CompilerParams(use_tc_tiling_on_sc=False),
    )


def _prop(y2, src2, dst):
    return _get_prop()(y2, src2, dst)


def _mm_body(x_ref, w_ref, o_ref):
    o_ref[...] = jnp.dot(x_ref[...], w_ref[...],
                         preferred_element_type=jnp.float32)


def _xproj(xs2, wxc):
    return pl.pallas_call(
        _mm_body,
        grid=(T * N // RB,),
        in_specs=[
            pl.BlockSpec((RB, D), lambda i: (i, 0)),
            pl.BlockSpec((D, 3 * D), lambda i: (0, 0)),
        ],
        out_specs=pl.BlockSpec((RB, 3 * D), lambda i: (i, 0)),
        out_shape=jax.ShapeDtypeStruct((T * N, 3 * D), jnp.float32),
    )(xs2, wxc)


def _dinv_body(dg_ref, o_ref):
    o_ref[...] = lax.rsqrt(dg_ref[0][:, 0:1])


def _dinv_t(o2deg, slab):
    return pl.pallas_call(
        _dinv_body,
        grid=(NB,),
        in_specs=[pl.BlockSpec((1, RB, HW), lambda i, slab=slab: (slab, i, 0))],
        out_specs=pl.BlockSpec((RB, 1), lambda i: (i, 0)),
        out_shape=jax.ShapeDtypeStruct((N, 1), jnp.float32),
    )(o2deg)


def _zr_body(xp_ref, h_ref, wh_ref, di_ref, oz_ref, or_ref):
    y = (xp_ref[...] + jnp.dot(h_ref[...], wh_ref[...],
                               preferred_element_type=jnp.float32)
         ) * di_ref[...]
    oz_ref[...] = jnp.stack([y[:, 0:HW], y[:, HW:D]], axis=0)
    or_ref[...] = jnp.stack([y[:, D:D + HW], y[:, D + HW:2 * D]], axis=0)


def _zr_pre(xp, h, whzr, dinv_t, t):
    return pl.pallas_call(
        _zr_body,
        grid=(NB,),
        in_specs=[
            pl.BlockSpec((RB, 2 * D), lambda i, t=t: (t * NB + i, 0)),
            pl.BlockSpec((RB, D), lambda i: (i, 0)),
            pl.BlockSpec((D, 2 * D), lambda i: (0, 0)),
            pl.BlockSpec((RB, 1), lambda i: (i, 0)),
        ],
        out_specs=[
            pl.BlockSpec((2, RB, HW), lambda i: (0, i, 0)),
            pl.BlockSpec((2, RB, HW), lambda i: (0, i, 0)),
        ],
        out_shape=[
            jax.ShapeDtypeStruct((2, NP, HW), jnp.float32),
            jax.ShapeDtypeStruct((2, NP, HW), jnp.float32),
        ],
    )(xp, h, whzr, dinv_t)


def _gates_body(pr0_ref, pr1_ref, di_ref, br_ref, xph_ref, h_ref, whh_ref,
                o_ref):
    di = di_ref[...]
    pr = jnp.concatenate([pr0_ref[0], pr1_ref[0]], axis=1)
    r = jax.nn.sigmoid(di * pr + br_ref[...])
    rh = r * h_ref[...]
    yh = (xph_ref[...] + jnp.dot(rh, whh_ref[...],
                                 preferred_element_type=jnp.float32)) * di
    o_ref[...] = jnp.stack([yh[:, :HW], yh[:, HW:]], axis=0)


def _gates(o2r, dinv_t, br, xp, h, whh, t):
    return pl.pallas_call(
        _gates_body,
        grid=(NB,),
        in_specs=[
            pl.BlockSpec((1, RB, HW), lambda i: (0, i, 0)),
            pl.BlockSpec((1, RB, HW), lambda i: (1, i, 0)),
            pl.BlockSpec((RB, 1), lambda i: (i, 0)),
            pl.BlockSpec((1, D), lambda i: (0, 0)),
            pl.BlockSpec((RB, D), lambda i, t=t: (t * NB + i, 2)),
            pl.BlockSpec((RB, D), lambda i: (i, 0)),
            pl.BlockSpec((D, D), lambda i: (0, 0)),
        ],
        out_specs=pl.BlockSpec((2, RB, HW), lambda i: (0, i, 0)),
        out_shape=jax.ShapeDtypeStruct((2, NP, HW), jnp.float32),
    )(o2r, o2r, dinv_t, br, xp, h, whh)


def _final_body(pz0_ref, pz1_ref, oh0_ref, oh1_ref, di_ref, bz_ref, bh_ref,
                h0_ref, o_ref):
    di = di_ref[...]
    pz = jnp.concatenate([pz0_ref[0], pz1_ref[0]], axis=1)
    z = jax.nn.sigmoid(di * pz + bz_ref[...])
    oh = jnp.concatenate([oh0_ref[0], oh1_ref[0]], axis=1)
    hh = jnp.tanh(di * oh + bh_ref[...])
    o_ref[...] = z * h0_ref[...] + (1.0 - z) * hh


def _final(o2z, o2h, dinv_t, bz, bh, h0):
    return pl.pallas_call(
        _final_body,
        grid=(NB,),
        in_specs=[
            pl.BlockSpec((1, RB, HW), lambda i: (0, i, 0)),
            pl.BlockSpec((1, RB, HW), lambda i: (1, i, 0)),
            pl.BlockSpec((1, RB, HW), lambda i: (0, i, 0)),
            pl.BlockSpec((1, RB, HW), lambda i: (1, i, 0)),
            pl.BlockSpec((RB, 1), lambda i: (i, 0)),
            pl.BlockSpec((1, D), lambda i: (0, 0)),
            pl.BlockSpec((1, D), lambda i: (0, 0)),
            pl.BlockSpec((1, D), lambda i: (0, 0)),
        ],
        out_specs=pl.BlockSpec((RB, D), lambda i: (i, 0)),
        out_shape=jax.ShapeDtypeStruct((N, D), jnp.float32),
    )(o2z, o2z, o2h, o2h, dinv_t, bz, bh, h0)


def kernel(xs, eis, W_xz, b_xz, W_hz, b_hz, W_xr, b_xr, W_hr, b_hr,
           W_xh, b_xh, W_hh, b_hh):
    xs2 = xs.reshape(T * N, D)
    pad = jnp.full((T, EPP - E), N, jnp.int32)
    src = jnp.concatenate([eis[:, 0, :], pad], axis=1)
    dst = jnp.concatenate([eis[:, 1, :], pad], axis=1)

    def pack(t0, t1):
        s2 = jnp.concatenate([src[t0], src[t1] + NP]).reshape(
            2 * NS * NCH, CH)
        d2 = jnp.concatenate([dst[t0], dst[t1]]).reshape(2 * NS * NCH, CH)
        return s2, d2

    wxc = jnp.concatenate([W_xz, W_xr, W_xh], axis=1)
    whzr = jnp.concatenate([W_hz, W_hr], axis=1)
    bz = (b_xz + b_hz).reshape(1, D)
    br = (b_xr + b_hr).reshape(1, D)
    bh = (b_xh + b_hh).reshape(1, D)

    xp = _xproj(xs2, wxc)
    ones_tab = jnp.ones((2 * NP, HW), jnp.float32)
    sA, dA = pack(0, 1)
    sB, dB = pack(2, 2)
    degA = _prop(ones_tab, sA, dA).reshape(2, NP, HW)
    degB = _prop(ones_tab, sB, dB).reshape(2, NP, HW)
    dinvs = [_dinv_t(degA, 0), _dinv_t(degA, 1), _dinv_t(degB, 0)]
    packs = [pack(t, t) for t in range(T)]

    h = jnp.zeros((N, D), jnp.float32)
    h0 = jnp.zeros((1, D), jnp.float32)
    outs = []
    for t in range(T):
        st, dt = packs[t]
        dinv_t = dinvs[t]
        yz, yr = _zr_pre(xp, h, whzr, dinv_t, t)
        o2z = _prop(yz.reshape(2 * NP, HW), st, dt).reshape(2, NP, HW)
        o2r = _prop(yr.reshape(2 * NP, HW), st, dt).reshape(2, NP, HW)
        yh2 = _gates(o2r, dinv_t, br, xp, h, W_hh, t)
        o2h = _prop(yh2.reshape(2 * NP, HW), st, dt).reshape(2, NP, HW)
        h = _final(o2z, o2h, dinv_t, bz, bh, h0)
        h0 = h[0:1]
        outs.append(h)
    return jnp.stack(outs)
